# baseline (speedup 1.0000x reference)
import functools

import jax
import jax.numpy as jnp
from jax import lax
from jax.experimental import pallas as pl
from jax.experimental.pallas import tpu as pltpu
from jax.experimental.pallas import tpu_sc as plsc

NUM_USERS = 10000
NUM_POIS = 10000
BATCH = 4096

NC = 2
NS = 16
L = 16
NW = NC * NS
BPW = BATCH // NW
G = 2
T = BPW // G
VECS = NUM_POIS // L
UNROLL = 5
NEG = -1000000000.0


def _process_row(buf, r, g_v):
    def it(j, acc):
        for u in range(UNROLL):
            off = (j * UNROLL + u) * L
            v = buf[r, pl.ds(off, L)]
            acc = acc + v
            buf[r, pl.ds(off, L)] = v + g_v[pl.ds(off, L)] * 0.001
        return acc

    acc = lax.fori_loop(0, VECS // UNROLL, it, jnp.zeros((L,), jnp.float32))
    nonzero_lanes = plsc.all_reduce_population_count(acc != 0.0)
    cold = nonzero_lanes[0] == 0

    @pl.when(cold)
    def _cold():
        def cp(j, c):
            for u in range(UNROLL):
                off = (j * UNROLL + u) * L
                buf[r, pl.ds(off, L)] = g_v[pl.ds(off, L)]
            return c

        lax.fori_loop(0, VECS // UNROLL, cp, 0)

    lane = lax.iota(jnp.int32, L)
    v0 = buf[r, pl.ds(0, L)]
    buf[r, pl.ds(0, L)] = jnp.where(lane == 0, NEG, v0)


def _body(table, uid, g_hbm, out, idx_v, g_v, buf, gsem, ssem):
    wid = lax.axis_index("s") * NC + lax.axis_index("c")
    base = wid * BPW

    pltpu.sync_copy(uid.at[wid], idx_v)
    pltpu.sync_copy(g_hbm, g_v)

    def group(t, _):
        pltpu.async_copy(table.at[idx_v.at[t]], buf, gsem).wait()
        for r in range(G):
            _process_row(buf, r, g_v)
        pltpu.async_copy(buf, out.at[pl.ds(base + t * G, G)], ssem).wait()
        return 0

    lax.fori_loop(0, T, group, 0)


_sc_call = functools.partial(
    pl.kernel,
    out_type=jax.ShapeDtypeStruct((BATCH, NUM_POIS), jnp.float32),
    mesh=plsc.VectorSubcoreMesh(
        core_axis_name="c", subcore_axis_name="s", num_cores=NC, num_subcores=NS
    ),
    scratch_types=[
        pltpu.VMEM((T, G), jnp.int32),
        pltpu.VMEM((NUM_POIS,), jnp.float32),
        pltpu.VMEM((G, NUM_POIS), jnp.float32),
        pltpu.SemaphoreType.DMA,
        pltpu.SemaphoreType.DMA,
    ],
    compiler_params=pltpu.CompilerParams(
        needs_layout_passes=False, use_tc_tiling_on_sc=False
    ),
)(_body)


def kernel(user_id, global_poi_cnt, user_poi_cnt):
    uid = user_id.astype(jnp.int32).reshape(NW, T, G)
    return _sc_call(user_poi_cnt, uid, global_poi_cnt)


# speedup vs baseline: 1.1750x; 1.1750x over previous
import functools

import jax
import jax.numpy as jnp
from jax import lax
from jax.experimental import pallas as pl
from jax.experimental.pallas import tpu as pltpu
from jax.experimental.pallas import tpu_sc as plsc

NUM_USERS = 10000
NUM_POIS = 10000
BATCH = 4096

NC = 2
NS = 16
L = 16
NW = NC * NS
BPW = BATCH // NW
G = 2
T = BPW // G
R = 4
VECS = NUM_POIS // L
UNROLL = 25
NEG = -1000000000.0


def _process_row(buf, s, r, g_v):
    def it(j, acc):
        for u in range(UNROLL):
            off = (j * UNROLL + u) * L
            v = buf[s, r, pl.ds(off, L)]
            acc = acc + v
            buf[s, r, pl.ds(off, L)] = v + g_v[pl.ds(off, L)] * 0.001
        return acc

    acc = lax.fori_loop(0, VECS // UNROLL, it, jnp.zeros((L,), jnp.float32))
    nonzero_lanes = plsc.all_reduce_population_count(acc != 0.0)
    cold = nonzero_lanes[0] == 0

    @pl.when(cold)
    def _cold():
        def cp(j, c):
            for u in range(UNROLL):
                off = (j * UNROLL + u) * L
                buf[s, r, pl.ds(off, L)] = g_v[pl.ds(off, L)]
            return c

        lax.fori_loop(0, VECS // UNROLL, cp, 0)

    lane = lax.iota(jnp.int32, L)
    v0 = buf[s, r, pl.ds(0, L)]
    buf[s, r, pl.ds(0, L)] = jnp.where(lane == 0, NEG, v0)


def _body(table, uid, g_hbm, out, idx_v, g_v, buf, *sems):
    gsems, ssems = sems[:R], sems[R:]
    wid = lax.axis_index("s") * NC + lax.axis_index("c")
    base = wid * BPW

    pltpu.sync_copy(uid.at[wid], idx_v)
    pltpu.sync_copy(g_hbm, g_v)

    def gather(t, slot):
        return pltpu.make_async_copy(
            table.at[idx_v.at[t]], buf.at[slot], gsems[slot]
        )

    def scatter(t, slot):
        return pltpu.make_async_copy(
            buf.at[slot], out.at[pl.ds(base + t * G, G)], ssems[slot]
        )

    for s in range(R - 1):
        gather(s, s).start()

    def block(tb, _):
        for s in range(R):
            t = tb * R + s
            gather(t, s).wait()
            for r in range(G):
                _process_row(buf, s, r, g_v)
            scatter(t, s).start()
            ps = (s + R - 1) % R

            @pl.when(t >= 1)
            def _drain():
                scatter(t - 1, ps).wait()

            @pl.when(t + R - 1 < T)
            def _next():
                gather(t + R - 1, ps).start()

        return 0

    lax.fori_loop(0, T // R, block, 0)
    scatter(T - 1, (T - 1) % R).wait()


_sc_call = functools.partial(
    pl.kernel,
    out_type=jax.ShapeDtypeStruct((BATCH, NUM_POIS), jnp.float32),
    mesh=plsc.VectorSubcoreMesh(
        core_axis_name="c", subcore_axis_name="s", num_cores=NC, num_subcores=NS
    ),
    scratch_types=[
        pltpu.VMEM((T, G), jnp.int32),
        pltpu.VMEM((NUM_POIS,), jnp.float32),
        pltpu.VMEM((R, G, NUM_POIS), jnp.float32),
    ]
    + [pltpu.SemaphoreType.DMA] * (2 * R),
    compiler_params=pltpu.CompilerParams(
        needs_layout_passes=False, use_tc_tiling_on_sc=False
    ),
)(_body)


def kernel(user_id, global_poi_cnt, user_poi_cnt):
    uid = user_id.astype(jnp.int32).reshape(NW, T, G)
    return _sc_call(user_poi_cnt, uid, global_poi_cnt)
